# initial kernel scaffold (unmeasured)
import jax
import jax.numpy as jnp
from jax import lax
from jax.experimental import pallas as pl
from jax.experimental.pallas import tpu as pltpu

N_DEV = 8


def kernel(x, w_mat):
    m_loc, k = x.shape
    _, n = w_mat.shape
    n_loc = n // N_DEV
    m_out = m_loc * N_DEV

    def body(x_ref, w_ref, out_ref, ysend_ref, yrecv_ref, amax_src_ref,
             amax_ref, send_sems, recv_sems, a_send_sems, a_recv_sems):
        me = lax.axis_index("i")

        barrier = pltpu.get_barrier_semaphore()
        for off in range(1, N_DEV):
            dst = (me + off) % N_DEV
            pl.semaphore_signal(
                barrier, inc=1,
                device_id=(dst,), device_id_type=pl.DeviceIdType.MESH,
            )
        pl.semaphore_wait(barrier, N_DEV - 1)

        amax = jnp.float32(0.0)

        rdmas = []
        for off in range(1, N_DEV):
            dst = (me + off) % N_DEV
            yblk = jnp.dot(
                x_ref[...],
                w_ref[:, pl.ds(dst * n_loc, n_loc)],
                preferred_element_type=jnp.float32,
            )
            amax = jnp.maximum(amax, jnp.max(jnp.abs(yblk)))
            ysend_ref[off - 1] = yblk
            rd = pltpu.make_async_remote_copy(
                src_ref=ysend_ref.at[off - 1],
                dst_ref=yrecv_ref.at[off - 1],
                send_sem=send_sems.at[off - 1],
                recv_sem=recv_sems.at[off - 1],
                device_id=(dst,),
                device_id_type=pl.DeviceIdType.MESH,
            )
            rd.start()
            rdmas.append(rd)

        yme = jnp.dot(
            x_ref[...],
            w_ref[:, pl.ds(me * n_loc, n_loc)],
            preferred_element_type=jnp.float32,
        )
        amax = jnp.maximum(amax, jnp.max(jnp.abs(yme)))
        ysend_ref[N_DEV - 1] = yme

        amax_src_ref[...] = jnp.full((1, 128), amax, jnp.float32)
        a_rdmas = []
        for off in range(1, N_DEV):
            dst = (me + off) % N_DEV
            ard = pltpu.make_async_remote_copy(
                src_ref=amax_src_ref,
                dst_ref=amax_ref.at[pl.ds(off - 1, 1)],
                send_sem=a_send_sems.at[off - 1],
                recv_sem=a_recv_sems.at[off - 1],
                device_id=(dst,),
                device_id_type=pl.DeviceIdType.MESH,
            )
            ard.start()
            a_rdmas.append(ard)

        for ard in a_rdmas:
            ard.wait_recv()
        for rd in rdmas:
            rd.wait_recv()

        gmax = jnp.maximum(amax, jnp.max(amax_ref[...]))
        scale = gmax / 448.0

        def quant_dequant(v):
            q = (v / scale).astype(jnp.float8_e4m3fn)
            return q.astype(jnp.float32) * scale

        out_ref[pl.ds(me * m_loc, m_loc), :] = quant_dequant(
            ysend_ref[N_DEV - 1]
        )
        for off in range(1, N_DEV):
            src = (me - off) % N_DEV
            out_ref[pl.ds(src * m_loc, m_loc), :] = quant_dequant(
                yrecv_ref[off - 1]
            )

        for rd in rdmas:
            rd.wait_send()
        for ard in a_rdmas:
            ard.wait_send()

    return pl.pallas_call(
        body,
        out_shape=jax.ShapeDtypeStruct((m_out, n_loc), jnp.float32),
        in_specs=[
            pl.BlockSpec(memory_space=pltpu.VMEM),
            pl.BlockSpec(memory_space=pltpu.VMEM),
        ],
        out_specs=pl.BlockSpec(memory_space=pltpu.VMEM),
        scratch_shapes=[
            pltpu.VMEM((N_DEV, m_loc, n_loc), jnp.float32),
            pltpu.VMEM((N_DEV - 1, m_loc, n_loc), jnp.float32),
            pltpu.VMEM((1, 128), jnp.float32),
            pltpu.VMEM((N_DEV - 1, 128), jnp.float32),
            pltpu.SemaphoreType.DMA((N_DEV - 1,)),
            pltpu.SemaphoreType.DMA((N_DEV - 1,)),
            pltpu.SemaphoreType.DMA((N_DEV - 1,)),
            pltpu.SemaphoreType.DMA((N_DEV - 1,)),
        ],
        compiler_params=pltpu.CompilerParams(collective_id=0),
    )(x, w_mat)


# baseline (device time: 58897 ns/iter reference)
import jax
import jax.numpy as jnp
from jax import lax
from jax.experimental import pallas as pl
from jax.experimental.pallas import tpu as pltpu

N_DEV = 8


def kernel(x, w_mat):
    m_loc, k = x.shape
    _, n = w_mat.shape
    n_loc = n // N_DEV
    m_out = m_loc * N_DEV

    def body(x_ref, w_ref, out_ref, ysend_ref, yrecv_ref, amax_src_ref,
             amax_ref, send_sems, recv_sems, a_send_sems, a_recv_sems):
        me = lax.axis_index("i")

        barrier = pltpu.get_barrier_semaphore()
        for off in range(1, N_DEV):
            dst = (me + off) % N_DEV
            pl.semaphore_signal(
                barrier, inc=1,
                device_id=(dst,), device_id_type=pl.DeviceIdType.MESH,
            )
        pl.semaphore_wait(barrier, N_DEV - 1)

        amax = jnp.float32(0.0)

        rdmas = []
        for off in range(1, N_DEV):
            dst = (me + off) % N_DEV
            yblk = jnp.dot(
                x_ref[...],
                w_ref[:, pl.ds(dst * n_loc, n_loc)],
                preferred_element_type=jnp.float32,
            )
            amax = jnp.maximum(amax, jnp.max(jnp.abs(yblk)))
            ysend_ref[off - 1] = yblk
            rd = pltpu.make_async_remote_copy(
                src_ref=ysend_ref.at[off - 1],
                dst_ref=yrecv_ref.at[off - 1],
                send_sem=send_sems.at[off - 1],
                recv_sem=recv_sems.at[off - 1],
                device_id=(dst,),
                device_id_type=pl.DeviceIdType.MESH,
            )
            rd.start()
            rdmas.append(rd)

        yme = jnp.dot(
            x_ref[...],
            w_ref[:, pl.ds(me * n_loc, n_loc)],
            preferred_element_type=jnp.float32,
        )
        amax = jnp.maximum(amax, jnp.max(jnp.abs(yme)))
        ysend_ref[N_DEV - 1] = yme

        amax_src_ref[...] = jnp.full((1, 128), amax, jnp.float32)
        a_rdmas = []
        for off in range(1, N_DEV):
            dst = (me + off) % N_DEV
            ard = pltpu.make_async_remote_copy(
                src_ref=amax_src_ref,
                dst_ref=amax_ref.at[pl.ds(off - 1, 1)],
                send_sem=a_send_sems.at[off - 1],
                recv_sem=a_recv_sems.at[off - 1],
                device_id=(dst,),
                device_id_type=pl.DeviceIdType.MESH,
            )
            ard.start()
            a_rdmas.append(ard)

        for ard in a_rdmas:
            ard.wait_recv()
        for rd in rdmas:
            rd.wait_recv()

        gmax = jnp.maximum(amax, jnp.max(amax_ref[...]))
        scale = gmax / 448.0

        def quant_dequant(v):
            q = (v / scale).astype(jnp.float8_e4m3fn)
            return q.astype(jnp.float32) * scale

        out_ref[pl.ds(me * m_loc, m_loc), :] = quant_dequant(
            ysend_ref[N_DEV - 1]
        )
        for off in range(1, N_DEV):
            src = (me - off) % N_DEV
            out_ref[pl.ds(src * m_loc, m_loc), :] = quant_dequant(
                yrecv_ref[off - 1]
            )

        for rd in rdmas:
            rd.wait_send()
        for ard in a_rdmas:
            ard.wait_send()

    return pl.pallas_call(
        body,
        out_shape=jax.ShapeDtypeStruct((m_out, n_loc), jnp.float32),
        in_specs=[
            pl.BlockSpec(memory_space=pltpu.VMEM),
            pl.BlockSpec(memory_space=pltpu.VMEM),
        ],
        out_specs=pl.BlockSpec(memory_space=pltpu.VMEM),
        scratch_shapes=[
            pltpu.VMEM((N_DEV, m_loc, n_loc), jnp.float32),
            pltpu.VMEM((N_DEV - 1, m_loc, n_loc), jnp.float32),
            pltpu.VMEM((1, 128), jnp.float32),
            pltpu.VMEM((N_DEV - 1, 128), jnp.float32),
            pltpu.SemaphoreType.DMA((N_DEV - 1,)),
            pltpu.SemaphoreType.DMA((N_DEV - 1,)),
            pltpu.SemaphoreType.DMA((N_DEV - 1,)),
            pltpu.SemaphoreType.DMA((N_DEV - 1,)),
        ],
        compiler_params=pltpu.CompilerParams(
            collective_id=0,
            vmem_limit_bytes=60 * 1024 * 1024,
        ),
    )(x, w_mat)


# device time: 26755 ns/iter; 2.2013x vs baseline; 2.2013x over previous
import jax
import jax.numpy as jnp
from jax import lax
from jax.experimental import pallas as pl
from jax.experimental.pallas import tpu as pltpu

N_DEV = 8


def kernel(x, w_mat):
    m_loc, k = x.shape
    _, n = w_mat.shape
    n_loc = n // N_DEV
    m_out = m_loc * N_DEV

    def body(x_ref, w_ref, out_ref, ysend_ref, yrecv_ref, amax_src_ref,
             amax_ref, send_sems, recv_sems, a_send_sems, a_recv_sems):
        me = lax.axis_index("i")

        barrier = pltpu.get_barrier_semaphore()
        for off in range(1, N_DEV):
            dst = (me + off) % N_DEV
            pl.semaphore_signal(
                barrier, inc=1,
                device_id=(dst,), device_id_type=pl.DeviceIdType.MESH,
            )
        pl.semaphore_wait(barrier, N_DEV - 1)

        amax = jnp.float32(0.0)

        xb = x_ref[...].astype(jnp.bfloat16)

        rdmas = []
        for off in range(1, N_DEV):
            dst = (me + off) % N_DEV
            yblk = jnp.dot(
                xb,
                w_ref[:, pl.ds(dst * n_loc, n_loc)].astype(jnp.bfloat16),
                preferred_element_type=jnp.float32,
            )
            amax = jnp.maximum(amax, jnp.max(jnp.abs(yblk)))
            ysend_ref[off - 1] = yblk
            rd = pltpu.make_async_remote_copy(
                src_ref=ysend_ref.at[off - 1],
                dst_ref=yrecv_ref.at[off - 1],
                send_sem=send_sems.at[off - 1],
                recv_sem=recv_sems.at[off - 1],
                device_id=(dst,),
                device_id_type=pl.DeviceIdType.MESH,
            )
            rd.start()
            rdmas.append(rd)

        yme = jnp.dot(
            xb,
            w_ref[:, pl.ds(me * n_loc, n_loc)].astype(jnp.bfloat16),
            preferred_element_type=jnp.float32,
        )
        amax = jnp.maximum(amax, jnp.max(jnp.abs(yme)))
        ysend_ref[N_DEV - 1] = yme

        amax_src_ref[...] = jnp.full((1, 128), amax, jnp.float32)
        a_rdmas = []
        for off in range(1, N_DEV):
            dst = (me + off) % N_DEV
            ard = pltpu.make_async_remote_copy(
                src_ref=amax_src_ref,
                dst_ref=amax_ref.at[pl.ds(off - 1, 1)],
                send_sem=a_send_sems.at[off - 1],
                recv_sem=a_recv_sems.at[off - 1],
                device_id=(dst,),
                device_id_type=pl.DeviceIdType.MESH,
            )
            ard.start()
            a_rdmas.append(ard)

        for ard in a_rdmas:
            ard.wait_recv()
        for rd in rdmas:
            rd.wait_recv()

        gmax = jnp.maximum(amax, jnp.max(amax_ref[...]))
        scale = gmax / 448.0

        def quant_dequant(v):
            q = (v / scale).astype(jnp.float8_e4m3fn)
            return q.astype(jnp.float32) * scale

        out_ref[pl.ds(me * m_loc, m_loc), :] = quant_dequant(
            ysend_ref[N_DEV - 1]
        )
        for off in range(1, N_DEV):
            src = (me - off) % N_DEV
            out_ref[pl.ds(src * m_loc, m_loc), :] = quant_dequant(
                yrecv_ref[off - 1]
            )

        for rd in rdmas:
            rd.wait_send()
        for ard in a_rdmas:
            ard.wait_send()

    return pl.pallas_call(
        body,
        out_shape=jax.ShapeDtypeStruct((m_out, n_loc), jnp.float32),
        in_specs=[
            pl.BlockSpec(memory_space=pltpu.VMEM),
            pl.BlockSpec(memory_space=pltpu.VMEM),
        ],
        out_specs=pl.BlockSpec(memory_space=pltpu.VMEM),
        scratch_shapes=[
            pltpu.VMEM((N_DEV, m_loc, n_loc), jnp.float32),
            pltpu.VMEM((N_DEV - 1, m_loc, n_loc), jnp.float32),
            pltpu.VMEM((1, 128), jnp.float32),
            pltpu.VMEM((N_DEV - 1, 128), jnp.float32),
            pltpu.SemaphoreType.DMA((N_DEV - 1,)),
            pltpu.SemaphoreType.DMA((N_DEV - 1,)),
            pltpu.SemaphoreType.DMA((N_DEV - 1,)),
            pltpu.SemaphoreType.DMA((N_DEV - 1,)),
        ],
        compiler_params=pltpu.CompilerParams(
            collective_id=0,
            vmem_limit_bytes=60 * 1024 * 1024,
        ),
    )(x, w_mat)
